# baseline (speedup 1.0000x reference)
import functools

import jax
import jax.numpy as jnp
from jax import lax
from jax.experimental import pallas as pl
from jax.experimental.pallas import tpu as pltpu
from jax.experimental.pallas import tpu_sc as plsc

_T = 701
_D = 128
_V = 13
_OA = 50
_L = 200
_HEAD = 256


def _expand(idx_ref, table_v, dst_ref, n_chunks):
    def body(c, _):
        s = pl.multiple_of(c * 16, 16)
        rv = idx_ref[pl.ds(s, 16)]
        for l in range(16):
            r = rv[l]
            t = s + l
            for k16 in range(_D // 16):
                sl = pl.ds(k16 * 16, 16)
                dst_ref[t, sl] = table_v[r, sl]
        return 0
    lax.fori_loop(0, n_chunks, body, 0)


def _out_dmas(out_hbm, b, h, cc, sem_h, sem_c, start):
    def cp(src, dst, sem):
        if start:
            pltpu.async_copy(src, dst, sem)
        else:
            pltpu.make_async_copy(src, dst, sem).wait()
    cp(h, out_hbm.at[b, pl.ds(0, _HEAD), :], sem_h)
    cp(cc, out_hbm.at[b, pl.ds(256, 320), :], sem_c)
    cp(cc.at[pl.ds(48, 120), :], out_hbm.at[b, pl.ds(576, 120), :], sem_c)
    cp(cc.at[pl.ds(48, 5), :], out_hbm.at[b, pl.ds(696, 5), :], sem_c)


def _make_sc(B):
    info = plsc.get_sparse_core_info()
    NC, NS = info.num_cores, info.num_subcores
    NW = NC * NS
    nb = B // NW
    mesh = plsc.VectorSubcoreMesh(core_axis_name="c", subcore_axis_name="s")

    @functools.partial(
        pl.kernel,
        out_type=jax.ShapeDtypeStruct((B, _T, _D), jnp.float32),
        mesh=mesh,
        compiler_params=pltpu.CompilerParams(needs_layout_passes=False),
        scratch_types=[
            pltpu.VMEM((_V, _D), jnp.float32),
            pltpu.VMEM((_HEAD, _D), jnp.float32),
            pltpu.VMEM((_HEAD, _D), jnp.float32),
            pltpu.VMEM((320, _D), jnp.float32),
            pltpu.VMEM((nb, _L), jnp.float32),
            pltpu.VMEM((nb, _OA), jnp.int32),
            pltpu.VMEM((256,), jnp.int32),
            pltpu.VMEM((320,), jnp.int32),
            pltpu.SemaphoreType.DMA,
            pltpu.SemaphoreType.DMA,
            pltpu.SemaphoreType.DMA,
            pltpu.SemaphoreType.DMA,
        ],
    )
    def k(aoat_hbm, lanes_hbm, emb_hbm, out_hbm,
          table_v, h0, h1, cc_v, lanes_v, aoat_v, idx_v, cidx_v,
          sem_h0, sem_h1, sem_c, sem_g):
        wid = lax.axis_index("s") * NC + lax.axis_index("c")
        b0 = wid * nb
        iota = lax.iota(jnp.int32, 16)

        pltpu.sync_copy(emb_hbm, table_v)
        pltpu.sync_copy(aoat_hbm.at[pl.ds(b0, nb), :], aoat_v)
        pltpu.sync_copy(lanes_hbm.at[pl.ds(b0, nb), :], lanes_v)

        for c in range(16):
            idx_v[pl.ds(c * 16, 16)] = jnp.zeros((16,), jnp.int32)
        idx_v[pl.ds(240, 16)] = jnp.where(iota >= 11, 10, 0)

        for c in range(20):
            p = c * 16 + iota
            cidx_v[pl.ds(c * 16, 16)] = jnp.where(
                p <= 44, 10, jnp.where((p - 45) % 2 == 0, 11, 12))
        _expand(cidx_v, table_v, cc_v, 20)

        def run_batch(i, h, sem):
            b = b0 + i
            for c in range(4):
                col = c * 16 + iota
                msk = col < _OA
                av = plsc.load_gather(aoat_v, [jnp.full((16,), i, jnp.int32), col],
                                      mask=msk)
                mapped = jnp.where(av == 3, 2,
                                   jnp.where(av == 14, 4,
                                             jnp.where(av == 12, 3, 1)))
                plsc.store_scatter(idx_v, [1 + col], mapped, mask=msk)
            for c in range(13):
                j = c * 16 + iota
                msk = j < _L
                tv = plsc.load_gather(lanes_v, [jnp.full((16,), i, jnp.int32), j],
                                      mask=msk)
                plsc.store_scatter(idx_v, [51 + j], tv.astype(jnp.int32) + 5,
                                   mask=msk)
            pltpu.async_copy(emb_hbm.at[idx_v], h, sem_g).wait()
            _out_dmas(out_hbm, b, h, cc_v, sem, sem_c, start=True)

        def outer(i2, _):
            @pl.when(i2 > 0)
            def _():
                pltpu.make_async_copy(h0, out_hbm.at[b0, pl.ds(0, _HEAD), :],
                                      sem_h0).wait()
            run_batch(2 * i2, h0, sem_h0)

            @pl.when(i2 > 0)
            def _():
                pltpu.make_async_copy(h1, out_hbm.at[b0, pl.ds(0, _HEAD), :],
                                      sem_h1).wait()
            run_batch(2 * i2 + 1, h1, sem_h1)
            return 0

        lax.fori_loop(0, nb // 2, outer, 0)
        pltpu.make_async_copy(h0, out_hbm.at[b0, pl.ds(0, _HEAD), :], sem_h0).wait()
        pltpu.make_async_copy(h1, out_hbm.at[b0, pl.ds(0, _HEAD), :], sem_h1).wait()

        def drain_const(i, _):
            def cp(src, dst):
                pltpu.make_async_copy(src, dst, sem_c).wait()
            cp(cc_v, out_hbm.at[b0, pl.ds(256, 320), :])
            cp(cc_v.at[pl.ds(48, 120), :], out_hbm.at[b0, pl.ds(576, 120), :])
            cp(cc_v.at[pl.ds(48, 5), :], out_hbm.at[b0, pl.ds(696, 5), :])
            return 0
        lax.fori_loop(0, nb, drain_const, 0)

    return k


@jax.jit
def _run_sc(aoat, tl_src, embedding):
    B = aoat.shape[0]
    return _make_sc(B)(aoat, tl_src, embedding)


def kernel(type, all_other_agents_types, lanes_mid, crosswalks, lanes, embedding):
    tl_src = lanes_mid[:, :, 0, -1]
    return _run_sc(all_other_agents_types, tl_src, embedding)


# speedup vs baseline: 6.9905x; 6.9905x over previous
import functools

import jax
import jax.numpy as jnp
from jax import lax
from jax.experimental import pallas as pl
from jax.experimental.pallas import tpu as pltpu
from jax.experimental.pallas import tpu_sc as plsc

_T = 701
_D = 128
_V = 13
_OA = 50
_L = 200
_HEAD = 256


def _expand(idx_ref, table_v, dst_ref, n_chunks):
    @plsc.parallel_loop(0, n_chunks * 16, step=16, unroll=2)
    def body(s):
        rv = idx_ref[pl.ds(pl.multiple_of(s, 16), 16)]
        for l in range(16):
            r = rv[l]
            t = s + l
            for k16 in range(_D // 16):
                sl = pl.ds(k16 * 16, 16)
                dst_ref[t, sl] = table_v[r, sl]


def _out_dmas(out_hbm, b, h, cc, sem_h, sem_c, start):
    def cp(src, dst, sem):
        if start:
            pltpu.async_copy(src, dst, sem)
        else:
            pltpu.make_async_copy(src, dst, sem).wait()
    cp(h, out_hbm.at[b, pl.ds(0, _HEAD), :], sem_h)
    cp(cc, out_hbm.at[b, pl.ds(256, 320), :], sem_c)
    cp(cc.at[pl.ds(48, 120), :], out_hbm.at[b, pl.ds(576, 120), :], sem_c)
    cp(cc.at[pl.ds(48, 5), :], out_hbm.at[b, pl.ds(696, 5), :], sem_c)


def _make_sc(B):
    info = plsc.get_sparse_core_info()
    NC, NS = info.num_cores, info.num_subcores
    NW = NC * NS
    nb = B // NW
    mesh = plsc.VectorSubcoreMesh(core_axis_name="c", subcore_axis_name="s")

    @functools.partial(
        pl.kernel,
        out_type=jax.ShapeDtypeStruct((B, _T, _D), jnp.float32),
        mesh=mesh,
        compiler_params=pltpu.CompilerParams(needs_layout_passes=False),
        scratch_types=[
            pltpu.VMEM((_V, _D), jnp.float32),
            pltpu.VMEM((_HEAD, _D), jnp.float32),
            pltpu.VMEM((_HEAD, _D), jnp.float32),
            pltpu.VMEM((320, _D), jnp.float32),
            pltpu.VMEM((nb, _L), jnp.float32),
            pltpu.VMEM((nb, _OA), jnp.int32),
            pltpu.VMEM((256,), jnp.int32),
            pltpu.VMEM((320,), jnp.int32),
            pltpu.SemaphoreType.DMA,
            pltpu.SemaphoreType.DMA,
            pltpu.SemaphoreType.DMA,
            pltpu.SemaphoreType.DMA,
        ],
    )
    def k(aoat_hbm, lanes_hbm, emb_hbm, out_hbm,
          table_v, h0, h1, cc_v, lanes_v, aoat_v, idx_v, cidx_v,
          sem_h0, sem_h1, sem_c, sem_g):
        wid = lax.axis_index("s") * NC + lax.axis_index("c")
        b0 = wid * nb
        iota = lax.iota(jnp.int32, 16)

        pltpu.sync_copy(emb_hbm, table_v)
        pltpu.sync_copy(aoat_hbm.at[pl.ds(b0, nb), :], aoat_v)
        pltpu.sync_copy(lanes_hbm.at[pl.ds(b0, nb), :], lanes_v)

        for c in range(16):
            idx_v[pl.ds(c * 16, 16)] = jnp.zeros((16,), jnp.int32)
        idx_v[pl.ds(240, 16)] = jnp.where(iota >= 11, 10, 0)

        for c in range(20):
            p = c * 16 + iota
            cidx_v[pl.ds(c * 16, 16)] = jnp.where(
                p <= 44, 10, jnp.where((p - 45) % 2 == 0, 11, 12))
        _expand(cidx_v, table_v, cc_v, 20)

        def run_batch(i, h, sem):
            b = b0 + i
            for c in range(4):
                col = c * 16 + iota
                msk = col < _OA
                av = plsc.load_gather(aoat_v, [jnp.full((16,), i, jnp.int32), col],
                                      mask=msk)
                mapped = jnp.where(av == 3, 2,
                                   jnp.where(av == 14, 4,
                                             jnp.where(av == 12, 3, 1)))
                plsc.store_scatter(idx_v, [1 + col], mapped, mask=msk)
            for c in range(13):
                j = c * 16 + iota
                msk = j < _L
                tv = plsc.load_gather(lanes_v, [jnp.full((16,), i, jnp.int32), j],
                                      mask=msk)
                plsc.store_scatter(idx_v, [51 + j], tv.astype(jnp.int32) + 5,
                                   mask=msk)
            _expand(idx_v, table_v, h, 16)
            _out_dmas(out_hbm, b, h, cc_v, sem, sem_c, start=True)

        def outer(i2, _):
            @pl.when(i2 > 0)
            def _():
                pltpu.make_async_copy(h0, out_hbm.at[b0, pl.ds(0, _HEAD), :],
                                      sem_h0).wait()
            run_batch(2 * i2, h0, sem_h0)

            @pl.when(i2 > 0)
            def _():
                pltpu.make_async_copy(h1, out_hbm.at[b0, pl.ds(0, _HEAD), :],
                                      sem_h1).wait()
            run_batch(2 * i2 + 1, h1, sem_h1)
            return 0

        lax.fori_loop(0, nb // 2, outer, 0)
        pltpu.make_async_copy(h0, out_hbm.at[b0, pl.ds(0, _HEAD), :], sem_h0).wait()
        pltpu.make_async_copy(h1, out_hbm.at[b0, pl.ds(0, _HEAD), :], sem_h1).wait()

        def drain_const(i, _):
            def cp(src, dst):
                pltpu.make_async_copy(src, dst, sem_c).wait()
            cp(cc_v, out_hbm.at[b0, pl.ds(256, 320), :])
            cp(cc_v.at[pl.ds(48, 120), :], out_hbm.at[b0, pl.ds(576, 120), :])
            cp(cc_v.at[pl.ds(48, 5), :], out_hbm.at[b0, pl.ds(696, 5), :])
            return 0
        lax.fori_loop(0, nb, drain_const, 0)

    return k


@jax.jit
def _run_sc(aoat, tl_src, embedding):
    B = aoat.shape[0]
    return _make_sc(B)(aoat, tl_src, embedding)


def kernel(type, all_other_agents_types, lanes_mid, crosswalks, lanes, embedding):
    tl_src = lanes_mid[:, :, 0, -1]
    return _run_sc(all_other_agents_types, tl_src, embedding)


# speedup vs baseline: 6.9955x; 1.0007x over previous
import functools

import jax
import jax.numpy as jnp
from jax import lax
from jax.experimental import pallas as pl
from jax.experimental.pallas import tpu as pltpu
from jax.experimental.pallas import tpu_sc as plsc

_T = 701
_D = 128
_V = 13
_OA = 50
_L = 200
_HEAD = 256


def _expand(idx_ref, table_v, dst_ref, n_chunks):
    @plsc.parallel_loop(0, n_chunks * 16, step=16, unroll=2)
    def body(s):
        rv = idx_ref[pl.ds(pl.multiple_of(s, 16), 16)]
        for l in range(16):
            r = rv[l]
            t = s + l
            for k16 in range(_D // 16):
                sl = pl.ds(k16 * 16, 16)
                dst_ref[t, sl] = table_v[r, sl]


def _out_dmas(out_hbm, b, h, cc, sem_h, sem_c, start):
    def cp(src, dst, sem):
        if start:
            pltpu.async_copy(src, dst, sem)
        else:
            pltpu.make_async_copy(src, dst, sem).wait()
    cp(h, out_hbm.at[b, pl.ds(0, _HEAD), :], sem_h)
    cp(cc, out_hbm.at[b, pl.ds(256, 320), :], sem_c)
    cp(cc.at[pl.ds(48, 120), :], out_hbm.at[b, pl.ds(576, 120), :], sem_c)
    cp(cc.at[pl.ds(48, 5), :], out_hbm.at[b, pl.ds(696, 5), :], sem_c)


def _make_sc(B):
    info = plsc.get_sparse_core_info()
    NC, NS = info.num_cores, info.num_subcores
    NW = NC * NS
    nb = B // NW
    mesh = plsc.VectorSubcoreMesh(core_axis_name="c", subcore_axis_name="s")

    @functools.partial(
        pl.kernel,
        out_type=jax.ShapeDtypeStruct((B, _T, _D), jnp.float32),
        mesh=mesh,
        compiler_params=pltpu.CompilerParams(needs_layout_passes=False),
        scratch_types=[
            pltpu.VMEM((_V, _D), jnp.float32),
            pltpu.VMEM((_HEAD, _D), jnp.float32),
            pltpu.VMEM((_HEAD, _D), jnp.float32),
            pltpu.VMEM((320, _D), jnp.float32),
            pltpu.VMEM((nb, _L), jnp.float32),
            pltpu.VMEM((nb, _OA), jnp.int32),
            pltpu.VMEM((256,), jnp.int32),
            pltpu.VMEM((320,), jnp.int32),
            pltpu.SemaphoreType.DMA,
            pltpu.SemaphoreType.DMA,
            pltpu.SemaphoreType.DMA,
        ],
    )
    def k(aoat_hbm, lanes_hbm, emb_hbm, out_hbm,
          table_v, h0, h1, cc_v, lanes_v, aoat_v, idx_v, cidx_v,
          sem_h0, sem_h1, sem_c):
        wid = lax.axis_index("s") * NC + lax.axis_index("c")
        b0 = wid * nb
        iota = lax.iota(jnp.int32, 16)

        pltpu.sync_copy(emb_hbm, table_v)
        pltpu.sync_copy(aoat_hbm.at[pl.ds(b0, nb), :], aoat_v)
        pltpu.sync_copy(lanes_hbm.at[pl.ds(b0, nb), :], lanes_v)

        for c in range(16):
            idx_v[pl.ds(c * 16, 16)] = jnp.zeros((16,), jnp.int32)
        idx_v[pl.ds(240, 16)] = jnp.where(iota >= 11, 10, 0)

        for c in range(20):
            p = c * 16 + iota
            cidx_v[pl.ds(c * 16, 16)] = jnp.where(
                p <= 44, 10, jnp.where((p - 45) % 2 == 0, 11, 12))
        _expand(cidx_v, table_v, cc_v, 20)

        def run_batch(i, h, sem):
            b = b0 + i
            for c in range(4):
                col = c * 16 + iota
                msk = col < _OA
                av = plsc.load_gather(aoat_v, [jnp.full((16,), i, jnp.int32), col],
                                      mask=msk)
                mapped = jnp.where(av == 3, 2,
                                   jnp.where(av == 14, 4,
                                             jnp.where(av == 12, 3, 1)))
                plsc.store_scatter(idx_v, [1 + col], mapped, mask=msk)
            for c in range(13):
                j = c * 16 + iota
                msk = j < _L
                tv = plsc.load_gather(lanes_v, [jnp.full((16,), i, jnp.int32), j],
                                      mask=msk)
                plsc.store_scatter(idx_v, [51 + j], tv.astype(jnp.int32) + 5,
                                   mask=msk)
            _expand(idx_v, table_v, h, 16)
            _out_dmas(out_hbm, b, h, cc_v, sem, sem_c, start=True)

        def outer(i2, _):
            @pl.when(i2 > 0)
            def _():
                pltpu.make_async_copy(h0, out_hbm.at[b0, pl.ds(0, _HEAD), :],
                                      sem_h0).wait()
            run_batch(2 * i2, h0, sem_h0)

            @pl.when(i2 > 0)
            def _():
                pltpu.make_async_copy(h1, out_hbm.at[b0, pl.ds(0, _HEAD), :],
                                      sem_h1).wait()
            run_batch(2 * i2 + 1, h1, sem_h1)
            return 0

        lax.fori_loop(0, nb // 2, outer, 0)
        pltpu.make_async_copy(h0, out_hbm.at[b0, pl.ds(0, _HEAD), :], sem_h0).wait()
        pltpu.make_async_copy(h1, out_hbm.at[b0, pl.ds(0, _HEAD), :], sem_h1).wait()

        def drain_const(i, _):
            def cp(src, dst):
                pltpu.make_async_copy(src, dst, sem_c).wait()
            cp(cc_v, out_hbm.at[b0, pl.ds(256, 320), :])
            cp(cc_v.at[pl.ds(48, 120), :], out_hbm.at[b0, pl.ds(576, 120), :])
            cp(cc_v.at[pl.ds(48, 5), :], out_hbm.at[b0, pl.ds(696, 5), :])
            return 0
        lax.fori_loop(0, nb, drain_const, 0)

    return k


@jax.jit
def _run_sc(aoat, tl_src, embedding):
    B = aoat.shape[0]
    return _make_sc(B)(aoat, tl_src, embedding)


def kernel(type, all_other_agents_types, lanes_mid, crosswalks, lanes, embedding):
    tl_src = lanes_mid[:, :, 0, -1]
    return _run_sc(all_other_agents_types, tl_src, embedding)
